# baseline (speedup 1.0000x reference)
import functools

import jax
import jax.numpy as jnp
from jax.experimental import pallas as pl
from jax.experimental.pallas import tpu as pltpu

_B, _L, _D, _EDIM, _VOCAB, _FR = 16, 2048, 256, 64, 50, 128
_TL = 512


def _leaky(x):
    return jnp.where(x >= 0, x, 0.01 * x)


def _side_body(h_ref, head_ref, tp_ref, W1_ref, b1_ref, Wc_ref, emb_ref,
               We_ref, frW_ref, frb_ref, out_ref):
    h = h_ref[0].astype(jnp.bfloat16)
    x = jnp.dot(h, W1_ref[:].astype(jnp.bfloat16),
                preferred_element_type=jnp.float32) + b1_ref[:]
    xg = _leaky(x).astype(jnp.bfloat16)
    AS = jnp.dot(xg, Wc_ref[:].astype(jnp.bfloat16),
                 preferred_element_type=jnp.float32)
    A = AS[:, :_D].astype(jnp.bfloat16)
    S = AS[:, _D:]
    T = jnp.dot(emb_ref[:].astype(jnp.bfloat16), We_ref[:].astype(jnp.bfloat16),
                preferred_element_type=jnp.float32).astype(jnp.bfloat16)
    tp = tp_ref[0, 0]
    ohe = (tp[:, None] == jax.lax.broadcasted_iota(jnp.int32, (_L, _VOCAB), 1)
           ).astype(jnp.bfloat16)
    SE = S + jnp.dot(ohe, T, preferred_element_type=jnp.float32)
    head = head_ref[0, 0]
    col_iota = jax.lax.broadcasted_iota(jnp.int32, (_TL, _L), 1)

    acc = jnp.zeros((1, _D), jnp.float32)
    for i in range(_L // _TL):
        hd = head[i * _TL:(i + 1) * _TL]
        oh = (hd[:, None] == col_iota).astype(jnp.bfloat16)
        g = jnp.dot(oh, A, preferred_element_type=jnp.float32)
        o = g + SE[i * _TL:(i + 1) * _TL, :]
        ss = jnp.sum(o * o, axis=1, keepdims=True)
        scale = jax.lax.rsqrt(jnp.maximum(ss, 1e-24))
        acc = acc + jnp.sum(o * scale, axis=0, keepdims=True)
    rep = acc * (1.0 / _L)
    r = jnp.dot(rep, frW_ref[:], preferred_element_type=jnp.float32) + frb_ref[:]
    out_ref[0] = _leaky(r)


def _run_side(h, head, tp, W1, b1, Wc, emb, We, frW, frb, interpret=False):
    head3 = head.astype(jnp.int32).reshape(_B, 1, _L)
    tp3 = tp.astype(jnp.int32).reshape(_B, 1, _L)
    full = lambda *s: pl.BlockSpec(s, lambda b: (0,) * len(s))
    return pl.pallas_call(
        _side_body,
        grid=(_B,),
        in_specs=[
            pl.BlockSpec((1, _L, _D), lambda b: (b, 0, 0)),
            pl.BlockSpec((1, 1, _L), lambda b: (b, 0, 0)),
            pl.BlockSpec((1, 1, _L), lambda b: (b, 0, 0)),
            full(_D, _D),
            full(1, _D),
            full(_D, 2 * _D),
            full(_VOCAB, _EDIM),
            full(_EDIM, _D),
            full(_D, _FR),
            full(1, _FR),
        ],
        out_specs=pl.BlockSpec((1, 1, _FR), lambda b: (b, 0, 0)),
        out_shape=jax.ShapeDtypeStruct((_B, 1, _FR), jnp.float32),
        compiler_params=pltpu.CompilerParams(
            dimension_semantics=("arbitrary",)),
        interpret=interpret,
    )(h, head3, tp3, W1, b1.reshape(1, _D), Wc, emb, We, frW,
      frb.reshape(1, _FR))


def kernel(h_x, x_mask, src_token_dense_mask, src_token_sparse_mask,
           src_token_inarc_type, src_token_arc_head, src_token_depth,
           src_word_inarc_type, src_word_inarc_type_mask, h_y, y_mask,
           tgt_token_dense_mask, tgt_token_sparse_mask, tgt_token_inarc_type,
           tgt_token_arc_head, tgt_token_depth, tgt_word_inarc_type,
           tgt_word_inarc_type_mask, src_W, src_b, tgt_W, tgt_b, inarc_emb,
           W_msg, W_edge, W_skip, fr_W, fr_b, interpret=False):
    Wc = jnp.concatenate([W_msg, W_skip], axis=1)
    src_tp = src_word_inarc_type * src_word_inarc_type_mask
    tgt_tp = tgt_word_inarc_type * tgt_word_inarc_type_mask
    src_repr = _run_side(h_x, src_token_arc_head, src_tp, src_W, src_b, Wc,
                         inarc_emb, W_edge, fr_W, fr_b, interpret=interpret)
    y_repr = _run_side(h_y, tgt_token_arc_head, tgt_tp, tgt_W, tgt_b, Wc,
                       inarc_emb, W_edge, fr_W, fr_b, interpret=interpret)
    return (src_repr, y_repr)


# speedup vs baseline: 1.2446x; 1.2446x over previous
import functools

import jax
import jax.numpy as jnp
from jax.experimental import pallas as pl
from jax.experimental.pallas import tpu as pltpu

_B, _L, _D, _EDIM, _VOCAB, _FR = 16, 2048, 256, 64, 50, 128
_TL = 256


def _leaky(x):
    return jnp.where(x >= 0, x, 0.01 * x)


def _side_body(h_ref, head_ref, tp_ref, W1_ref, b1_ref, Wc_ref, emb_ref,
               We_ref, frW_ref, frb_ref, out_ref):
    h = h_ref[0].astype(jnp.bfloat16)
    x = jnp.dot(h, W1_ref[:].astype(jnp.bfloat16),
                preferred_element_type=jnp.float32) + b1_ref[:]
    xg = _leaky(x).astype(jnp.bfloat16)
    AS = jnp.dot(xg, Wc_ref[:].astype(jnp.bfloat16),
                 preferred_element_type=jnp.float32)
    A = AS[:, :_D].astype(jnp.bfloat16)
    S = AS[:, _D:]
    T = jnp.dot(emb_ref[:], We_ref[:], preferred_element_type=jnp.float32)
    tp = tp_ref[0, 0]
    ohe = (tp[:, None] == jax.lax.broadcasted_iota(jnp.int32, (_L, _VOCAB), 1)
           ).astype(jnp.float32)
    SE = S + jnp.dot(ohe, T, preferred_element_type=jnp.float32)
    head = head_ref[0, 0]
    col_iota = jax.lax.broadcasted_iota(jnp.int32, (_TL, _L), 1)

    acc = jnp.zeros((1, _D), jnp.float32)
    for i in range(_L // _TL):
        hd = head[i * _TL:(i + 1) * _TL]
        oh = (hd[:, None] == col_iota).astype(jnp.bfloat16)
        g = jnp.dot(oh, A, preferred_element_type=jnp.float32)
        o = g + SE[i * _TL:(i + 1) * _TL, :]
        ss = jnp.sum(o * o, axis=1, keepdims=True)
        scale = jax.lax.rsqrt(jnp.maximum(ss, 1e-24))
        acc = acc + jnp.sum(o * scale, axis=0, keepdims=True)
    rep = acc * (1.0 / _L)
    r = jnp.dot(rep, frW_ref[:], preferred_element_type=jnp.float32) + frb_ref[:]
    out_ref[0] = _leaky(r)


def _run_side(h, head, tp, W1, b1, Wc, emb, We, frW, frb, interpret=False):
    head3 = head.astype(jnp.int32).reshape(_B, 1, _L)
    tp3 = tp.astype(jnp.int32).reshape(_B, 1, _L)
    full = lambda *s: pl.BlockSpec(s, lambda b: (0,) * len(s))
    return pl.pallas_call(
        _side_body,
        grid=(_B,),
        in_specs=[
            pl.BlockSpec((1, _L, _D), lambda b: (b, 0, 0)),
            pl.BlockSpec((1, 1, _L), lambda b: (b, 0, 0)),
            pl.BlockSpec((1, 1, _L), lambda b: (b, 0, 0)),
            full(_D, _D),
            full(1, _D),
            full(_D, 2 * _D),
            full(_VOCAB, _EDIM),
            full(_EDIM, _D),
            full(_D, _FR),
            full(1, _FR),
        ],
        out_specs=pl.BlockSpec((1, 1, _FR), lambda b: (b, 0, 0)),
        out_shape=jax.ShapeDtypeStruct((_B, 1, _FR), jnp.float32),
        compiler_params=pltpu.CompilerParams(
            dimension_semantics=("arbitrary",)),
        interpret=interpret,
    )(h, head3, tp3, W1, b1.reshape(1, _D), Wc, emb, We, frW,
      frb.reshape(1, _FR))


def kernel(h_x, x_mask, src_token_dense_mask, src_token_sparse_mask,
           src_token_inarc_type, src_token_arc_head, src_token_depth,
           src_word_inarc_type, src_word_inarc_type_mask, h_y, y_mask,
           tgt_token_dense_mask, tgt_token_sparse_mask, tgt_token_inarc_type,
           tgt_token_arc_head, tgt_token_depth, tgt_word_inarc_type,
           tgt_word_inarc_type_mask, src_W, src_b, tgt_W, tgt_b, inarc_emb,
           W_msg, W_edge, W_skip, fr_W, fr_b, interpret=False):
    Wc = jnp.concatenate([W_msg, W_skip], axis=1)
    src_tp = src_word_inarc_type * src_word_inarc_type_mask
    tgt_tp = tgt_word_inarc_type * tgt_word_inarc_type_mask
    src_repr = _run_side(h_x, src_token_arc_head, src_tp, src_W, src_b, Wc,
                         inarc_emb, W_edge, fr_W, fr_b, interpret=interpret)
    y_repr = _run_side(h_y, tgt_token_arc_head, tgt_tp, tgt_W, tgt_b, Wc,
                       inarc_emb, W_edge, fr_W, fr_b, interpret=interpret)
    return (src_repr, y_repr)


# speedup vs baseline: 1.2475x; 1.0024x over previous
import functools

import jax
import jax.numpy as jnp
from jax.experimental import pallas as pl
from jax.experimental.pallas import tpu as pltpu

_B, _L, _D, _EDIM, _VOCAB, _FR = 16, 2048, 256, 64, 50, 128
_TL = 512


def _leaky(x):
    return jnp.where(x >= 0, x, 0.01 * x)


def _side_body(h_ref, head_ref, tp_ref, W1_ref, b1_ref, Wc_ref, emb_ref,
               We_ref, frW_ref, frb_ref, out_ref):
    h = h_ref[0].astype(jnp.bfloat16)
    x = jnp.dot(h, W1_ref[:].astype(jnp.bfloat16),
                preferred_element_type=jnp.float32) + b1_ref[:]
    xg = _leaky(x).astype(jnp.bfloat16)
    AS = jnp.dot(xg, Wc_ref[:].astype(jnp.bfloat16),
                 preferred_element_type=jnp.float32)
    A = AS[:, :_D].astype(jnp.bfloat16)
    S = AS[:, _D:]
    T = jnp.dot(emb_ref[:], We_ref[:], preferred_element_type=jnp.float32)
    tp = tp_ref[0, 0]
    ohe = (tp[:, None] == jax.lax.broadcasted_iota(jnp.int32, (_L, _VOCAB), 1)
           ).astype(jnp.float32)
    SE = S + jnp.dot(ohe, T, preferred_element_type=jnp.float32)
    head = head_ref[0, 0]
    col_iota = jax.lax.broadcasted_iota(jnp.int32, (_TL, _L), 1)

    acc = jnp.zeros((1, _D), jnp.float32)
    for i in range(_L // _TL):
        hd = head[i * _TL:(i + 1) * _TL]
        oh = (hd[:, None] == col_iota).astype(jnp.bfloat16)
        g = jnp.dot(oh, A, preferred_element_type=jnp.float32)
        o = g + SE[i * _TL:(i + 1) * _TL, :]
        ss = jnp.sum(o * o, axis=1, keepdims=True)
        scale = jax.lax.rsqrt(jnp.maximum(ss, 1e-24))
        acc = acc + jnp.sum(o * scale, axis=0, keepdims=True)
    rep = acc * (1.0 / _L)
    r = jnp.dot(rep, frW_ref[:], preferred_element_type=jnp.float32) + frb_ref[:]
    out_ref[0] = _leaky(r)


def _run_side(h, head, tp, W1, b1, Wc, emb, We, frW, frb, interpret=False):
    head3 = head.astype(jnp.int32).reshape(_B, 1, _L)
    tp3 = tp.astype(jnp.int32).reshape(_B, 1, _L)
    full = lambda *s: pl.BlockSpec(s, lambda b: (0,) * len(s))
    return pl.pallas_call(
        _side_body,
        grid=(_B,),
        in_specs=[
            pl.BlockSpec((1, _L, _D), lambda b: (b, 0, 0)),
            pl.BlockSpec((1, 1, _L), lambda b: (b, 0, 0)),
            pl.BlockSpec((1, 1, _L), lambda b: (b, 0, 0)),
            full(_D, _D),
            full(1, _D),
            full(_D, 2 * _D),
            full(_VOCAB, _EDIM),
            full(_EDIM, _D),
            full(_D, _FR),
            full(1, _FR),
        ],
        out_specs=pl.BlockSpec((1, 1, _FR), lambda b: (b, 0, 0)),
        out_shape=jax.ShapeDtypeStruct((_B, 1, _FR), jnp.float32),
        compiler_params=pltpu.CompilerParams(
            dimension_semantics=("parallel",)),
        interpret=interpret,
    )(h, head3, tp3, W1, b1.reshape(1, _D), Wc, emb, We, frW,
      frb.reshape(1, _FR))


def kernel(h_x, x_mask, src_token_dense_mask, src_token_sparse_mask,
           src_token_inarc_type, src_token_arc_head, src_token_depth,
           src_word_inarc_type, src_word_inarc_type_mask, h_y, y_mask,
           tgt_token_dense_mask, tgt_token_sparse_mask, tgt_token_inarc_type,
           tgt_token_arc_head, tgt_token_depth, tgt_word_inarc_type,
           tgt_word_inarc_type_mask, src_W, src_b, tgt_W, tgt_b, inarc_emb,
           W_msg, W_edge, W_skip, fr_W, fr_b, interpret=False):
    Wc = jnp.concatenate([W_msg, W_skip], axis=1)
    src_tp = src_word_inarc_type * src_word_inarc_type_mask
    tgt_tp = tgt_word_inarc_type * tgt_word_inarc_type_mask
    src_repr = _run_side(h_x, src_token_arc_head, src_tp, src_W, src_b, Wc,
                         inarc_emb, W_edge, fr_W, fr_b, interpret=interpret)
    y_repr = _run_side(h_y, tgt_token_arc_head, tgt_tp, tgt_W, tgt_b, Wc,
                       inarc_emb, W_edge, fr_W, fr_b, interpret=interpret)
    return (src_repr, y_repr)
